# baseline (speedup 1.0000x reference)
import functools

import jax
import jax.numpy as jnp
from jax import lax
from jax.experimental import pallas as pl
from jax.experimental.pallas import tpu as pltpu
from jax.experimental.pallas import tpu_sc as plsc

N = 8192
D = 256
ANS_NUM = 2048
ANS_SIZE = 20
TEMP = 0.07


_ENT_R = 512


def _ent_body(mem_ref, rows_ref, out_ref, logT, uref):
    a = mem_ref[...]
    r = rows_ref[...]
    l = lax.dot_general(a, r, (((1,), (1,)), ((), ())),
                        preferred_element_type=jnp.float32)
    logT[...] = l / TEMP
    m = jnp.max(logT[...], axis=0, keepdims=True)

    t0 = jnp.exp(logT[pl.ds(0, 8), :] - m)
    uref[pl.ds(0, 8), :] = t0

    def loop1(j, acc):
        t = jnp.exp(logT[pl.ds(8 * j, 8), :] - m)
        uref[pl.ds(8 * j, 8), :] = t
        return acc + t

    acc = lax.fori_loop(1, N // 8, loop1, t0, unroll=8)
    z4 = acc[0:4] + acc[4:8]
    z2 = z4[0:2] + z4[2:4]
    Z = z2[0:1] + z2[1:2]

    p0 = uref[pl.ds(0, 8), :] / Z
    s0 = p0 * jnp.log(p0)

    def loop2(j, s):
        p = uref[pl.ds(8 * j, 8), :] / Z
        return s + p * jnp.log(p)

    s = lax.fori_loop(1, N // 8, loop2, s0, unroll=8)
    s4 = s[0:4] + s[4:8]
    s2 = s4[0:2] + s4[2:4]
    S = s2[0:1] + s2[1:2]
    out_ref[...] = (-S).reshape(1, 1, _ENT_R)


_ent_call = pl.pallas_call(
    _ent_body,
    grid=(N // _ENT_R,),
    in_specs=[pl.BlockSpec((N, D), lambda i: (0, 0)),
              pl.BlockSpec((_ENT_R, D), lambda i: (i, 0))],
    out_specs=pl.BlockSpec((1, 1, _ENT_R), lambda i: (i, 0, 0)),
    out_shape=jax.ShapeDtypeStruct((N // _ENT_R, 1, _ENT_R), jnp.float32),
    scratch_shapes=[pltpu.VMEM((N, _ENT_R), jnp.float32),
                    pltpu.VMEM((N, _ENT_R), jnp.float32)],
)


_RK_BLK = 512


def _rank_body(kcol_ref, krow_ref, out_ref):
    kc = kcol_ref[...]
    kr = krow_ref[...].reshape(1, _RK_BLK)
    i0 = pl.program_id(0) * _RK_BLK
    jio = lax.broadcasted_iota(jnp.int32, (N, 1), 0)
    iio = i0 + lax.broadcasted_iota(jnp.int32, (1, _RK_BLK), 1)
    gt = (kc > kr).astype(jnp.int32)
    tie = ((kc == kr) & (jio < iio)).astype(jnp.int32)
    rank = jnp.sum(gt + tie, axis=0)
    out_ref[...] = rank.reshape(1, 1, _RK_BLK)


_rank_call = pl.pallas_call(
    _rank_body,
    grid=(N // _RK_BLK,),
    in_specs=[pl.BlockSpec((N, 1), lambda i: (0, 0)),
              pl.BlockSpec((1, 1, _RK_BLK), lambda i: (i, 0, 0))],
    out_specs=pl.BlockSpec((1, 1, _RK_BLK), lambda i: (i, 0, 0)),
    out_shape=jax.ShapeDtypeStruct((N // _RK_BLK, 1, _RK_BLK), jnp.int32),
)


_NB_BLK = 512


def _nb_body(afeat_ref, mem_ref, aidx_ref, combo_ref, out_ref, cnt_ref):
    _CBIG = jnp.int32(1 << 24)
    a = afeat_ref[...]
    memf = mem_ref[...]
    aidx = aidx_ref[...]
    combo = combo_ref[...]
    s = lax.dot_general(a, memf, (((1,), (1,)), ((), ())),
                        preferred_element_type=jnp.float32)
    cio = lax.broadcasted_iota(jnp.int32, (_NB_BLK, N), 1)
    self_hit = cio == aidx
    acm = jnp.min(jnp.where(self_hit, combo, _CBIG), axis=1, keepdims=True)
    alab = jnp.bitwise_and(acm, jnp.int32(1023))
    s = jnp.where(self_hit, jnp.float32(-1.0), s)
    cols = []
    labs = []
    for _ in range(ANS_SIZE):
        m = jnp.max(s, axis=1, keepdims=True)
        cmin = jnp.min(jnp.where(s == m, combo, _CBIG), axis=1, keepdims=True)
        cols.append(jnp.right_shift(cmin, 10))
        labs.append(jnp.bitwise_and(cmin, jnp.int32(1023)))
        s = jnp.where(combo == cmin, jnp.float32(-2.0), s)
    pad = jnp.zeros((_NB_BLK, 32 - ANS_SIZE), jnp.int32)
    out_ref[...] = jnp.concatenate(cols + [pad], axis=1)
    nlab = jnp.concatenate(labs, axis=1)
    cnt = jnp.sum((nlab == alab).astype(jnp.float32))
    lane = lax.broadcasted_iota(jnp.int32, (1, 128), 1)
    cnt_ref[...] = jnp.where(lane == 0, cnt, jnp.float32(0.0)).reshape(1, 1, 128)


_nb_call = pl.pallas_call(
    _nb_body,
    grid=(ANS_NUM // _NB_BLK,),
    in_specs=[pl.BlockSpec((_NB_BLK, D), lambda i: (i, 0)),
              pl.BlockSpec((N, D), lambda i: (0, 0)),
              pl.BlockSpec((_NB_BLK, 1), lambda i: (i, 0)),
              pl.BlockSpec((1, N), lambda i: (0, 0))],
    out_specs=[pl.BlockSpec((_NB_BLK, 32), lambda i: (i, 0)),
               pl.BlockSpec((1, 1, 128), lambda i: (i, 0, 0))],
    out_shape=[jax.ShapeDtypeStruct((ANS_NUM, 32), jnp.int32),
               jax.ShapeDtypeStruct((ANS_NUM // _NB_BLK, 1, 128), jnp.float32)],
)


_SC_NC = 2
_SC_NS = 16
_SC_NW = _SC_NC * _SC_NS
_A_PER_W = ANS_NUM // _SC_NW


def _gather_body(mem_ref, aidx_ref, out_ref, idx_v, rows_v, sem):
    wid = lax.axis_index("s") * _SC_NC + lax.axis_index("c")
    base = wid * _A_PER_W
    pltpu.sync_copy(aidx_ref.at[pl.ds(base, _A_PER_W)], idx_v)
    pltpu.async_copy(mem_ref.at[idx_v], rows_v, sem).wait()
    pltpu.sync_copy(rows_v, out_ref.at[pl.ds(base, _A_PER_W)])


@functools.lru_cache(maxsize=None)
def _sc_gather_kernel():
    mesh = plsc.VectorSubcoreMesh(core_axis_name="c", subcore_axis_name="s")
    return pl.kernel(
        _gather_body,
        out_type=jax.ShapeDtypeStruct((ANS_NUM, D), jnp.float32),
        mesh=mesh,
        scratch_types=[
            pltpu.VMEM((_A_PER_W,), jnp.int32),
            pltpu.VMEM((_A_PER_W, D), jnp.float32),
            pltpu.SemaphoreType.DMA,
        ],
    )


def kernel(memory, cheat_labels, round):
    ent = _ent_call(memory, memory).reshape(N)

    key = -(ent * round)
    rank = _rank_call(key.reshape(N, 1),
                      key.reshape(N // _RK_BLK, 1, _RK_BLK)).reshape(N)

    idx_all = jnp.arange(N, dtype=jnp.int32)
    is_anchor = rank < ANS_NUM
    anchor_indexes = jnp.zeros(ANS_NUM, jnp.int32).at[
        jnp.where(is_anchor, rank, ANS_NUM)].set(idx_all, mode="drop")
    inst_rank = jnp.cumsum((~is_anchor).astype(jnp.int32)).astype(jnp.int32)
    position = jnp.where(is_anchor, rank, -inst_rank)
    instance_indexes = jnp.zeros(N - ANS_NUM, jnp.int32).at[
        jnp.where(is_anchor, N - ANS_NUM, inst_rank - 1)].set(idx_all, mode="drop")

    anchor_feat = _sc_gather_kernel()(memory, anchor_indexes)

    combo = (jnp.arange(N, dtype=jnp.int32) * 1024 + cheat_labels).reshape(1, N)
    nb, cnts = _nb_call(anchor_feat, memory,
                        anchor_indexes.reshape(ANS_NUM, 1), combo)
    neighbours = nb[:, :ANS_SIZE]
    consistency = (jnp.sum(cnts) / jnp.float32(ANS_NUM * ANS_SIZE)).reshape(())

    return ent, anchor_indexes, instance_indexes, position, neighbours, consistency


# speedup vs baseline: 1.0008x; 1.0008x over previous
import functools

import jax
import jax.numpy as jnp
from jax import lax
from jax.experimental import pallas as pl
from jax.experimental.pallas import tpu as pltpu
from jax.experimental.pallas import tpu_sc as plsc

N = 8192
D = 256
ANS_NUM = 2048
ANS_SIZE = 20
TEMP = 0.07


_ENT_R = 512


def _ent_body(mem_ref, rows_ref, out_ref, logT, uref):
    a = mem_ref[...]
    r = rows_ref[...]
    l = lax.dot_general(a, r, (((1,), (1,)), ((), ())),
                        preferred_element_type=jnp.float32)
    logT[...] = l / TEMP
    m = jnp.max(logT[...], axis=0, keepdims=True)

    t0 = jnp.exp(logT[pl.ds(0, 8), :] - m)
    uref[pl.ds(0, 8), :] = t0

    def loop1(j, acc):
        t = jnp.exp(logT[pl.ds(8 * j, 8), :] - m)
        uref[pl.ds(8 * j, 8), :] = t
        return acc + t

    acc = lax.fori_loop(1, N // 8, loop1, t0, unroll=8)
    z4 = acc[0:4] + acc[4:8]
    z2 = z4[0:2] + z4[2:4]
    Z = z2[0:1] + z2[1:2]

    p0 = uref[pl.ds(0, 8), :] / Z
    s0 = p0 * jnp.log(p0)

    def loop2(j, s):
        p = uref[pl.ds(8 * j, 8), :] / Z
        return s + p * jnp.log(p)

    s = lax.fori_loop(1, N // 8, loop2, s0, unroll=8)
    s4 = s[0:4] + s[4:8]
    s2 = s4[0:2] + s4[2:4]
    S = s2[0:1] + s2[1:2]
    out_ref[...] = (-S).reshape(1, 1, _ENT_R)


_ent_call = pl.pallas_call(
    _ent_body,
    grid=(N // _ENT_R,),
    in_specs=[pl.BlockSpec((N, D), lambda i: (0, 0)),
              pl.BlockSpec((_ENT_R, D), lambda i: (i, 0))],
    out_specs=pl.BlockSpec((1, 1, _ENT_R), lambda i: (i, 0, 0)),
    out_shape=jax.ShapeDtypeStruct((N // _ENT_R, 1, _ENT_R), jnp.float32),
    scratch_shapes=[pltpu.VMEM((N, _ENT_R), jnp.float32),
                    pltpu.VMEM((N, _ENT_R), jnp.float32)],
)


_RK_BLK = 1024


def _rank_body(kcol_ref, krow_ref, out_ref):
    kc = kcol_ref[...]
    kr = krow_ref[...].reshape(1, _RK_BLK)
    i0 = pl.program_id(0) * _RK_BLK
    jio = lax.broadcasted_iota(jnp.int32, (N, 1), 0)
    iio = i0 + lax.broadcasted_iota(jnp.int32, (1, _RK_BLK), 1)
    gt = (kc > kr).astype(jnp.int32)
    tie = ((kc == kr) & (jio < iio)).astype(jnp.int32)
    rank = jnp.sum(gt + tie, axis=0)
    out_ref[...] = rank.reshape(1, 1, _RK_BLK)


_rank_call = pl.pallas_call(
    _rank_body,
    grid=(N // _RK_BLK,),
    in_specs=[pl.BlockSpec((N, 1), lambda i: (0, 0)),
              pl.BlockSpec((1, 1, _RK_BLK), lambda i: (i, 0, 0))],
    out_specs=pl.BlockSpec((1, 1, _RK_BLK), lambda i: (i, 0, 0)),
    out_shape=jax.ShapeDtypeStruct((N // _RK_BLK, 1, _RK_BLK), jnp.int32),
)


_NB_BLK = 512


def _nb_body(afeat_ref, mem_ref, aidx_ref, combo_ref, out_ref, cnt_ref):
    _CBIG = jnp.int32(1 << 24)
    a = afeat_ref[...]
    memf = mem_ref[...]
    aidx = aidx_ref[...]
    combo = combo_ref[...]
    s = lax.dot_general(a, memf, (((1,), (1,)), ((), ())),
                        preferred_element_type=jnp.float32)
    cio = lax.broadcasted_iota(jnp.int32, (_NB_BLK, N), 1)
    self_hit = cio == aidx
    acm = jnp.min(jnp.where(self_hit, combo, _CBIG), axis=1, keepdims=True)
    alab = jnp.bitwise_and(acm, jnp.int32(1023))
    s = jnp.where(self_hit, jnp.float32(-1.0), s)
    cols = []
    labs = []
    for _ in range(ANS_SIZE):
        m = jnp.max(s, axis=1, keepdims=True)
        cmin = jnp.min(jnp.where(s == m, combo, _CBIG), axis=1, keepdims=True)
        cols.append(jnp.right_shift(cmin, 10))
        labs.append(jnp.bitwise_and(cmin, jnp.int32(1023)))
        s = jnp.where(combo == cmin, jnp.float32(-2.0), s)
    pad = jnp.zeros((_NB_BLK, 32 - ANS_SIZE), jnp.int32)
    out_ref[...] = jnp.concatenate(cols + [pad], axis=1)
    nlab = jnp.concatenate(labs, axis=1)
    cnt = jnp.sum((nlab == alab).astype(jnp.float32))
    lane = lax.broadcasted_iota(jnp.int32, (1, 128), 1)
    cnt_ref[...] = jnp.where(lane == 0, cnt, jnp.float32(0.0)).reshape(1, 1, 128)


_nb_call = pl.pallas_call(
    _nb_body,
    grid=(ANS_NUM // _NB_BLK,),
    in_specs=[pl.BlockSpec((_NB_BLK, D), lambda i: (i, 0)),
              pl.BlockSpec((N, D), lambda i: (0, 0)),
              pl.BlockSpec((_NB_BLK, 1), lambda i: (i, 0)),
              pl.BlockSpec((1, N), lambda i: (0, 0))],
    out_specs=[pl.BlockSpec((_NB_BLK, 32), lambda i: (i, 0)),
               pl.BlockSpec((1, 1, 128), lambda i: (i, 0, 0))],
    out_shape=[jax.ShapeDtypeStruct((ANS_NUM, 32), jnp.int32),
               jax.ShapeDtypeStruct((ANS_NUM // _NB_BLK, 1, 128), jnp.float32)],
)


_SC_NC = 2
_SC_NS = 16
_SC_NW = _SC_NC * _SC_NS
_A_PER_W = ANS_NUM // _SC_NW


def _gather_body(mem_ref, aidx_ref, out_ref, idx_v, rows_v, sem):
    wid = lax.axis_index("s") * _SC_NC + lax.axis_index("c")
    base = wid * _A_PER_W
    pltpu.sync_copy(aidx_ref.at[pl.ds(base, _A_PER_W)], idx_v)
    pltpu.async_copy(mem_ref.at[idx_v], rows_v, sem).wait()
    pltpu.sync_copy(rows_v, out_ref.at[pl.ds(base, _A_PER_W)])


@functools.lru_cache(maxsize=None)
def _sc_gather_kernel():
    mesh = plsc.VectorSubcoreMesh(core_axis_name="c", subcore_axis_name="s")
    return pl.kernel(
        _gather_body,
        out_type=jax.ShapeDtypeStruct((ANS_NUM, D), jnp.float32),
        mesh=mesh,
        scratch_types=[
            pltpu.VMEM((_A_PER_W,), jnp.int32),
            pltpu.VMEM((_A_PER_W, D), jnp.float32),
            pltpu.SemaphoreType.DMA,
        ],
    )


def kernel(memory, cheat_labels, round):
    ent = _ent_call(memory, memory).reshape(N)

    key = -(ent * round)
    rank = _rank_call(key.reshape(N, 1),
                      key.reshape(N // _RK_BLK, 1, _RK_BLK)).reshape(N)

    idx_all = jnp.arange(N, dtype=jnp.int32)
    is_anchor = rank < ANS_NUM
    anchor_indexes = jnp.zeros(ANS_NUM, jnp.int32).at[
        jnp.where(is_anchor, rank, ANS_NUM)].set(idx_all, mode="drop")
    inst_rank = jnp.cumsum((~is_anchor).astype(jnp.int32)).astype(jnp.int32)
    position = jnp.where(is_anchor, rank, -inst_rank)
    instance_indexes = jnp.zeros(N - ANS_NUM, jnp.int32).at[
        jnp.where(is_anchor, N - ANS_NUM, inst_rank - 1)].set(idx_all, mode="drop")

    anchor_feat = _sc_gather_kernel()(memory, anchor_indexes)

    combo = (jnp.arange(N, dtype=jnp.int32) * 1024 + cheat_labels).reshape(1, N)
    nb, cnts = _nb_call(anchor_feat, memory,
                        anchor_indexes.reshape(ANS_NUM, 1), combo)
    neighbours = nb[:, :ANS_SIZE]
    consistency = (jnp.sum(cnts) / jnp.float32(ANS_NUM * ANS_SIZE)).reshape(())

    return ent, anchor_indexes, instance_indexes, position, neighbours, consistency


# speedup vs baseline: 1.0439x; 1.0431x over previous
import functools

import jax
import jax.numpy as jnp
from jax import lax
from jax.experimental import pallas as pl
from jax.experimental.pallas import tpu as pltpu
from jax.experimental.pallas import tpu_sc as plsc

N = 8192
D = 256
ANS_NUM = 2048
ANS_SIZE = 20
TEMP = 0.07


_ENT_R = 512


def _ent_body(mem_ref, rows_ref, out_ref, logT, uref):
    a = mem_ref[...]
    r = rows_ref[...]
    l = lax.dot_general(a, r, (((1,), (1,)), ((), ())),
                        preferred_element_type=jnp.float32)
    logT[...] = l / TEMP
    m = jnp.max(logT[...], axis=0, keepdims=True)

    t0 = jnp.exp(logT[pl.ds(0, 8), :] - m)
    uref[pl.ds(0, 8), :] = t0

    def loop1(j, acc):
        t = jnp.exp(logT[pl.ds(8 * j, 8), :] - m)
        uref[pl.ds(8 * j, 8), :] = t
        return acc + t

    acc = lax.fori_loop(1, N // 8, loop1, t0, unroll=16)
    z4 = acc[0:4] + acc[4:8]
    z2 = z4[0:2] + z4[2:4]
    Z = z2[0:1] + z2[1:2]

    p0 = uref[pl.ds(0, 8), :] / Z
    s0 = p0 * jnp.log(p0)

    def loop2(j, s):
        p = uref[pl.ds(8 * j, 8), :] / Z
        return s + p * jnp.log(p)

    s = lax.fori_loop(1, N // 8, loop2, s0, unroll=16)
    s4 = s[0:4] + s[4:8]
    s2 = s4[0:2] + s4[2:4]
    S = s2[0:1] + s2[1:2]
    out_ref[...] = (-S).reshape(1, 1, _ENT_R)


_ent_call = pl.pallas_call(
    _ent_body,
    grid=(N // _ENT_R,),
    in_specs=[pl.BlockSpec((N, D), lambda i: (0, 0)),
              pl.BlockSpec((_ENT_R, D), lambda i: (i, 0))],
    out_specs=pl.BlockSpec((1, 1, _ENT_R), lambda i: (i, 0, 0)),
    out_shape=jax.ShapeDtypeStruct((N // _ENT_R, 1, _ENT_R), jnp.float32),
    scratch_shapes=[pltpu.VMEM((N, _ENT_R), jnp.float32),
                    pltpu.VMEM((N, _ENT_R), jnp.float32)],
)


_RK_BLK = 1024


def _rank_body(kcol_ref, krow_ref, out_ref):
    kc = kcol_ref[...]
    kr = krow_ref[...].reshape(1, _RK_BLK)
    i0 = pl.program_id(0) * _RK_BLK
    jio = lax.broadcasted_iota(jnp.int32, (N, 1), 0)
    iio = i0 + lax.broadcasted_iota(jnp.int32, (1, _RK_BLK), 1)
    gt = (kc > kr).astype(jnp.int32)
    tie = ((kc == kr) & (jio < iio)).astype(jnp.int32)
    rank = jnp.sum(gt + tie, axis=0)
    out_ref[...] = rank.reshape(1, 1, _RK_BLK)


_rank_call = pl.pallas_call(
    _rank_body,
    grid=(N // _RK_BLK,),
    in_specs=[pl.BlockSpec((N, 1), lambda i: (0, 0)),
              pl.BlockSpec((1, 1, _RK_BLK), lambda i: (i, 0, 0))],
    out_specs=pl.BlockSpec((1, 1, _RK_BLK), lambda i: (i, 0, 0)),
    out_shape=jax.ShapeDtypeStruct((N // _RK_BLK, 1, _RK_BLK), jnp.int32),
)


_NB_BLK = 512


def _nb_body(afeat_ref, mem_ref, aidx_ref, combo_ref, out_ref, cnt_ref):
    _CBIG = jnp.int32(1 << 24)
    a = afeat_ref[...]
    memf = mem_ref[...]
    aidx = aidx_ref[...]
    combo = combo_ref[...]
    s = lax.dot_general(a, memf, (((1,), (1,)), ((), ())),
                        preferred_element_type=jnp.float32)
    cio = lax.broadcasted_iota(jnp.int32, (_NB_BLK, N), 1)
    self_hit = cio == aidx
    acm = jnp.min(jnp.where(self_hit, combo, _CBIG), axis=1, keepdims=True)
    alab = jnp.bitwise_and(acm, jnp.int32(1023))
    s = jnp.where(self_hit, jnp.float32(-1.0), s)
    cols = []
    labs = []
    for _ in range(ANS_SIZE):
        m = jnp.max(s, axis=1, keepdims=True)
        cmin = jnp.min(jnp.where(s == m, combo, _CBIG), axis=1, keepdims=True)
        cols.append(jnp.right_shift(cmin, 10))
        labs.append(jnp.bitwise_and(cmin, jnp.int32(1023)))
        s = jnp.where(combo == cmin, jnp.float32(-2.0), s)
    pad = jnp.zeros((_NB_BLK, 32 - ANS_SIZE), jnp.int32)
    out_ref[...] = jnp.concatenate(cols + [pad], axis=1)
    nlab = jnp.concatenate(labs, axis=1)
    cnt = jnp.sum((nlab == alab).astype(jnp.float32))
    lane = lax.broadcasted_iota(jnp.int32, (1, 128), 1)
    cnt_ref[...] = jnp.where(lane == 0, cnt, jnp.float32(0.0)).reshape(1, 1, 128)


_nb_call = pl.pallas_call(
    _nb_body,
    grid=(ANS_NUM // _NB_BLK,),
    in_specs=[pl.BlockSpec((_NB_BLK, D), lambda i: (i, 0)),
              pl.BlockSpec((N, D), lambda i: (0, 0)),
              pl.BlockSpec((_NB_BLK, 1), lambda i: (i, 0)),
              pl.BlockSpec((1, N), lambda i: (0, 0))],
    out_specs=[pl.BlockSpec((_NB_BLK, 32), lambda i: (i, 0)),
               pl.BlockSpec((1, 1, 128), lambda i: (i, 0, 0))],
    out_shape=[jax.ShapeDtypeStruct((ANS_NUM, 32), jnp.int32),
               jax.ShapeDtypeStruct((ANS_NUM // _NB_BLK, 1, 128), jnp.float32)],
)


_SC_NC = 2
_SC_NS = 16
_SC_NW = _SC_NC * _SC_NS
_A_PER_W = ANS_NUM // _SC_NW


def _gather_body(mem_ref, aidx_ref, out_ref, idx_v, rows_v, sem):
    wid = lax.axis_index("s") * _SC_NC + lax.axis_index("c")
    base = wid * _A_PER_W
    pltpu.sync_copy(aidx_ref.at[pl.ds(base, _A_PER_W)], idx_v)
    pltpu.async_copy(mem_ref.at[idx_v], rows_v, sem).wait()
    pltpu.sync_copy(rows_v, out_ref.at[pl.ds(base, _A_PER_W)])


@functools.lru_cache(maxsize=None)
def _sc_gather_kernel():
    mesh = plsc.VectorSubcoreMesh(core_axis_name="c", subcore_axis_name="s")
    return pl.kernel(
        _gather_body,
        out_type=jax.ShapeDtypeStruct((ANS_NUM, D), jnp.float32),
        mesh=mesh,
        scratch_types=[
            pltpu.VMEM((_A_PER_W,), jnp.int32),
            pltpu.VMEM((_A_PER_W, D), jnp.float32),
            pltpu.SemaphoreType.DMA,
        ],
    )


def kernel(memory, cheat_labels, round):
    ent = _ent_call(memory, memory).reshape(N)

    key = -(ent * round)
    rank = _rank_call(key.reshape(N, 1),
                      key.reshape(N // _RK_BLK, 1, _RK_BLK)).reshape(N)

    idx_all = jnp.arange(N, dtype=jnp.int32)
    is_anchor = rank < ANS_NUM
    anchor_indexes = jnp.zeros(ANS_NUM, jnp.int32).at[
        jnp.where(is_anchor, rank, ANS_NUM)].set(idx_all, mode="drop")
    inst_rank = jnp.cumsum((~is_anchor).astype(jnp.int32)).astype(jnp.int32)
    position = jnp.where(is_anchor, rank, -inst_rank)
    instance_indexes = jnp.zeros(N - ANS_NUM, jnp.int32).at[
        jnp.where(is_anchor, N - ANS_NUM, inst_rank - 1)].set(idx_all, mode="drop")

    anchor_feat = _sc_gather_kernel()(memory, anchor_indexes)

    combo = (jnp.arange(N, dtype=jnp.int32) * 1024 + cheat_labels).reshape(1, N)
    nb, cnts = _nb_call(anchor_feat, memory,
                        anchor_indexes.reshape(ANS_NUM, 1), combo)
    neighbours = nb[:, :ANS_SIZE]
    consistency = (jnp.sum(cnts) / jnp.float32(ANS_NUM * ANS_SIZE)).reshape(())

    return ent, anchor_indexes, instance_indexes, position, neighbours, consistency


# speedup vs baseline: 1.0470x; 1.0029x over previous
import functools

import jax
import jax.numpy as jnp
from jax import lax
from jax.experimental import pallas as pl
from jax.experimental.pallas import tpu as pltpu
from jax.experimental.pallas import tpu_sc as plsc

N = 8192
D = 256
ANS_NUM = 2048
ANS_SIZE = 20
TEMP = 0.07


_ENT_R = 512


def _ent_body(mem_ref, rows_ref, out_ref, logT, uref):
    a = mem_ref[...]
    r = rows_ref[...]
    l = lax.dot_general(a, r, (((1,), (1,)), ((), ())),
                        preferred_element_type=jnp.float32)
    logT[...] = l / TEMP
    m = jnp.max(logT[...], axis=0, keepdims=True)

    t0 = jnp.exp(logT[pl.ds(0, 8), :] - m)
    uref[pl.ds(0, 8), :] = t0

    def loop1(j, acc):
        t = jnp.exp(logT[pl.ds(8 * j, 8), :] - m)
        uref[pl.ds(8 * j, 8), :] = t
        return acc + t

    acc = lax.fori_loop(1, N // 8, loop1, t0, unroll=32)
    z4 = acc[0:4] + acc[4:8]
    z2 = z4[0:2] + z4[2:4]
    Z = z2[0:1] + z2[1:2]

    p0 = uref[pl.ds(0, 8), :] / Z
    s0 = p0 * jnp.log(p0)

    def loop2(j, s):
        p = uref[pl.ds(8 * j, 8), :] / Z
        return s + p * jnp.log(p)

    s = lax.fori_loop(1, N // 8, loop2, s0, unroll=32)
    s4 = s[0:4] + s[4:8]
    s2 = s4[0:2] + s4[2:4]
    S = s2[0:1] + s2[1:2]
    out_ref[...] = (-S).reshape(1, 1, _ENT_R)


_ent_call = pl.pallas_call(
    _ent_body,
    grid=(N // _ENT_R,),
    in_specs=[pl.BlockSpec((N, D), lambda i: (0, 0)),
              pl.BlockSpec((_ENT_R, D), lambda i: (i, 0))],
    out_specs=pl.BlockSpec((1, 1, _ENT_R), lambda i: (i, 0, 0)),
    out_shape=jax.ShapeDtypeStruct((N // _ENT_R, 1, _ENT_R), jnp.float32),
    scratch_shapes=[pltpu.VMEM((N, _ENT_R), jnp.float32),
                    pltpu.VMEM((N, _ENT_R), jnp.float32)],
)


_RK_BLK = 1024


def _rank_body(kcol_ref, krow_ref, out_ref):
    kc = kcol_ref[...]
    kr = krow_ref[...].reshape(1, _RK_BLK)
    i0 = pl.program_id(0) * _RK_BLK
    jio = lax.broadcasted_iota(jnp.int32, (N, 1), 0)
    iio = i0 + lax.broadcasted_iota(jnp.int32, (1, _RK_BLK), 1)
    gt = (kc > kr).astype(jnp.int32)
    tie = ((kc == kr) & (jio < iio)).astype(jnp.int32)
    rank = jnp.sum(gt + tie, axis=0)
    out_ref[...] = rank.reshape(1, 1, _RK_BLK)


_rank_call = pl.pallas_call(
    _rank_body,
    grid=(N // _RK_BLK,),
    in_specs=[pl.BlockSpec((N, 1), lambda i: (0, 0)),
              pl.BlockSpec((1, 1, _RK_BLK), lambda i: (i, 0, 0))],
    out_specs=pl.BlockSpec((1, 1, _RK_BLK), lambda i: (i, 0, 0)),
    out_shape=jax.ShapeDtypeStruct((N // _RK_BLK, 1, _RK_BLK), jnp.int32),
)


_NB_BLK = 512


def _nb_body(afeat_ref, mem_ref, aidx_ref, combo_ref, out_ref, cnt_ref):
    _CBIG = jnp.int32(1 << 24)
    a = afeat_ref[...]
    memf = mem_ref[...]
    aidx = aidx_ref[...]
    combo = combo_ref[...]
    s = lax.dot_general(a, memf, (((1,), (1,)), ((), ())),
                        preferred_element_type=jnp.float32)
    cio = lax.broadcasted_iota(jnp.int32, (_NB_BLK, N), 1)
    self_hit = cio == aidx
    acm = jnp.min(jnp.where(self_hit, combo, _CBIG), axis=1, keepdims=True)
    alab = jnp.bitwise_and(acm, jnp.int32(1023))
    s = jnp.where(self_hit, jnp.float32(-1.0), s)
    cols = []
    labs = []
    for _ in range(ANS_SIZE):
        m = jnp.max(s, axis=1, keepdims=True)
        cmin = jnp.min(jnp.where(s == m, combo, _CBIG), axis=1, keepdims=True)
        cols.append(jnp.right_shift(cmin, 10))
        labs.append(jnp.bitwise_and(cmin, jnp.int32(1023)))
        s = jnp.where(combo == cmin, jnp.float32(-2.0), s)
    pad = jnp.zeros((_NB_BLK, 32 - ANS_SIZE), jnp.int32)
    out_ref[...] = jnp.concatenate(cols + [pad], axis=1)
    nlab = jnp.concatenate(labs, axis=1)
    cnt = jnp.sum((nlab == alab).astype(jnp.float32))
    lane = lax.broadcasted_iota(jnp.int32, (1, 128), 1)
    cnt_ref[...] = jnp.where(lane == 0, cnt, jnp.float32(0.0)).reshape(1, 1, 128)


_nb_call = pl.pallas_call(
    _nb_body,
    grid=(ANS_NUM // _NB_BLK,),
    in_specs=[pl.BlockSpec((_NB_BLK, D), lambda i: (i, 0)),
              pl.BlockSpec((N, D), lambda i: (0, 0)),
              pl.BlockSpec((_NB_BLK, 1), lambda i: (i, 0)),
              pl.BlockSpec((1, N), lambda i: (0, 0))],
    out_specs=[pl.BlockSpec((_NB_BLK, 32), lambda i: (i, 0)),
               pl.BlockSpec((1, 1, 128), lambda i: (i, 0, 0))],
    out_shape=[jax.ShapeDtypeStruct((ANS_NUM, 32), jnp.int32),
               jax.ShapeDtypeStruct((ANS_NUM // _NB_BLK, 1, 128), jnp.float32)],
)


_SC_NC = 2
_SC_NS = 16
_SC_NW = _SC_NC * _SC_NS
_A_PER_W = ANS_NUM // _SC_NW


def _gather_body(mem_ref, aidx_ref, out_ref, idx_v, rows_v, sem):
    wid = lax.axis_index("s") * _SC_NC + lax.axis_index("c")
    base = wid * _A_PER_W
    pltpu.sync_copy(aidx_ref.at[pl.ds(base, _A_PER_W)], idx_v)
    pltpu.async_copy(mem_ref.at[idx_v], rows_v, sem).wait()
    pltpu.sync_copy(rows_v, out_ref.at[pl.ds(base, _A_PER_W)])


@functools.lru_cache(maxsize=None)
def _sc_gather_kernel():
    mesh = plsc.VectorSubcoreMesh(core_axis_name="c", subcore_axis_name="s")
    return pl.kernel(
        _gather_body,
        out_type=jax.ShapeDtypeStruct((ANS_NUM, D), jnp.float32),
        mesh=mesh,
        scratch_types=[
            pltpu.VMEM((_A_PER_W,), jnp.int32),
            pltpu.VMEM((_A_PER_W, D), jnp.float32),
            pltpu.SemaphoreType.DMA,
        ],
    )


def kernel(memory, cheat_labels, round):
    ent = _ent_call(memory, memory).reshape(N)

    key = -(ent * round)
    rank = _rank_call(key.reshape(N, 1),
                      key.reshape(N // _RK_BLK, 1, _RK_BLK)).reshape(N)

    idx_all = jnp.arange(N, dtype=jnp.int32)
    is_anchor = rank < ANS_NUM
    anchor_indexes = jnp.zeros(ANS_NUM, jnp.int32).at[
        jnp.where(is_anchor, rank, ANS_NUM)].set(idx_all, mode="drop")
    inst_rank = jnp.cumsum((~is_anchor).astype(jnp.int32)).astype(jnp.int32)
    position = jnp.where(is_anchor, rank, -inst_rank)
    instance_indexes = jnp.zeros(N - ANS_NUM, jnp.int32).at[
        jnp.where(is_anchor, N - ANS_NUM, inst_rank - 1)].set(idx_all, mode="drop")

    anchor_feat = _sc_gather_kernel()(memory, anchor_indexes)

    combo = (jnp.arange(N, dtype=jnp.int32) * 1024 + cheat_labels).reshape(1, N)
    nb, cnts = _nb_call(anchor_feat, memory,
                        anchor_indexes.reshape(ANS_NUM, 1), combo)
    neighbours = nb[:, :ANS_SIZE]
    consistency = (jnp.sum(cnts) / jnp.float32(ANS_NUM * ANS_SIZE)).reshape(())

    return ent, anchor_indexes, instance_indexes, position, neighbours, consistency
